# baseline (speedup 1.0000x reference)
import jax
import jax.numpy as jnp
from jax import lax
from jax.experimental import pallas as pl
from jax.experimental.pallas import tpu as pltpu
from jax.experimental.pallas import tpu_sc as plsc

_N_MEM = 16
_L = 16
_B_PER_W = 32
_N_INPUTS = 128
_N_RULES = 512
_POS = _N_INPUTS * _N_MEM
_CSTRIDE = _POS + 8
_OSTRIDE = _N_RULES + 8
_MAGIC = 2.0 ** 23


def _splat(v, l):
    return jnp.take(v, jnp.full((_L,), l, jnp.int32))


def _sc_body(fx_hbm, selt_hbm, out_hbm, chunk_lo, chunk_hi, sel_v, out_v):
    wid = lax.axis_index("s") * 2 + lax.axis_index("c")
    b0 = wid * _B_PER_W
    for r in range(_L):
        pltpu.sync_copy(fx_hbm.at[pl.ds((b0 + r) * _POS, _POS)],
                        chunk_lo.at[pl.ds(r * _CSTRIDE, _POS)])
        pltpu.sync_copy(fx_hbm.at[pl.ds((b0 + _L + r) * _POS, _POS)],
                        chunk_hi.at[pl.ds(r * _CSTRIDE, _POS)])
    lane = lax.iota(jnp.int32, _L)
    ones = jnp.full((_L,), 1.0, jnp.float32)
    plsc.store_scatter(chunk_lo, [lane * _CSTRIDE + _POS], ones)
    plsc.store_scatter(chunk_hi, [lane * _CSTRIDE + _POS], ones)
    rb_lo = lane * _CSTRIDE
    ob_lo = lane * _OSTRIDE
    ob_hi = (lane + _L) * _OSTRIDE

    def quarter_body(q, carry):
        pltpu.sync_copy(selt_hbm.at[pl.ds(q * 128, 128), :], sel_v)

        def rule_body(rl, carry2):
            pvecs = []
            for g in range(8):
                y = sel_v[rl, pl.ds(g * _L, _L)] * jnp.float32(_N_MEM)
                m = ((y + jnp.float32(_MAGIC)) - jnp.float32(_MAGIC)).astype(jnp.int32)
                i_base = (lane + g * _L) * _N_MEM
                pvecs.append(jnp.where(m < _N_MEM, i_base + m, _POS))
            acc = [jnp.full((_L,), 1.0, jnp.float32) for _ in range(8)]
            for g in range(8):
                for l in range(_L):
                    a = rb_lo + _splat(pvecs[g], l)
                    glo = plsc.load_gather(chunk_lo, [a])
                    ghi = plsc.load_gather(chunk_hi, [a])
                    k = l % 4
                    acc[k] = acc[k] * glo
                    acc[4 + k] = acc[4 + k] * ghi
            lo = (acc[0] * acc[1]) * (acc[2] * acc[3])
            hi = (acc[4] * acc[5]) * (acc[6] * acc[7])
            r_idx = jnp.full((_L,), q * 128 + rl, jnp.int32)
            plsc.store_scatter(out_v, [ob_lo + r_idx], lo)
            plsc.store_scatter(out_v, [ob_hi + r_idx], hi)
            return carry2

        lax.fori_loop(0, 128, rule_body, 0)
        return carry

    lax.fori_loop(0, _N_RULES // 128, quarter_body, 0)
    for r in range(_B_PER_W):
        pltpu.sync_copy(out_v.at[pl.ds(r * _OSTRIDE, _N_RULES)],
                        out_hbm.at[pl.ds((b0 + r) * _N_RULES, _N_RULES)])


def kernel(fuzzified_x, input_selectors):
    b = fuzzified_x.shape[0]
    fx_flat = fuzzified_x.reshape(b * _POS)
    sel_t = input_selectors.T
    mesh = plsc.VectorSubcoreMesh(core_axis_name="c", subcore_axis_name="s")
    f = pl.kernel(
        _sc_body,
        out_type=jax.ShapeDtypeStruct((b * _N_RULES,), jnp.float32),
        mesh=mesh,
        compiler_params=pltpu.CompilerParams(needs_layout_passes=False),
        scratch_types=[
            pltpu.VMEM((_L * _CSTRIDE,), jnp.float32),
            pltpu.VMEM((_L * _CSTRIDE,), jnp.float32),
            pltpu.VMEM((128, _N_INPUTS), jnp.float32),
            pltpu.VMEM((_B_PER_W * _OSTRIDE,), jnp.float32),
        ],
    )
    return f(fx_flat, sel_t).reshape(b, _N_RULES)


# speedup vs baseline: 1.0688x; 1.0688x over previous
import jax
import jax.numpy as jnp
from jax import lax
from jax.experimental import pallas as pl
from jax.experimental.pallas import tpu as pltpu
from jax.experimental.pallas import tpu_sc as plsc

_N_MEM = 16
_L = 16
_B_PER_W = 32
_N_INPUTS = 128
_N_RULES = 512
_POS = _N_INPUTS * _N_MEM
_CSTRIDE = _POS + 8
_OSTRIDE = _N_RULES + 8
_MAGIC = 2.0 ** 23


def _splat(v, l):
    return jnp.take(v, jnp.full((_L,), l, jnp.int32))


def _sc_body(fx_hbm, selt_hbm, out_hbm, chunk_v, sel_v, out_v):
    wid = lax.axis_index("s") * 2 + lax.axis_index("c")
    b0 = wid * _B_PER_W
    for r in range(_B_PER_W):
        pltpu.sync_copy(fx_hbm.at[pl.ds((b0 + r) * _POS, _POS)],
                        chunk_v.at[pl.ds(r * _CSTRIDE, _POS)])
    lane = lax.iota(jnp.int32, _L)
    ones = jnp.full((_L,), 1.0, jnp.float32)
    plsc.store_scatter(chunk_v, [lane * _CSTRIDE + _POS], ones)
    plsc.store_scatter(chunk_v, [(lane + _L) * _CSTRIDE + _POS], ones)
    half = jnp.full((_L,), 0.5, jnp.float32)
    rb_lo = lane * _CSTRIDE
    rb_hi = (lane + _L) * _CSTRIDE
    ob_lo = lane * _OSTRIDE
    ob_hi = (lane + _L) * _OSTRIDE

    def quarter_body(q, carry):
        pltpu.sync_copy(selt_hbm.at[pl.ds(q * 128, 128), :], sel_v)

        def rule_body(rl, carry2):
            pvecs = []
            for g in range(8):
                y = sel_v[rl, pl.ds(g * _L, _L)] * jnp.float32(_N_MEM)
                m = ((y + jnp.float32(_MAGIC)) - jnp.float32(_MAGIC)).astype(jnp.int32)
                i_base = (lane + g * _L) * _N_MEM
                pvecs.append(jnp.where(m < _N_MEM, i_base + m, _POS))
            acc = [jnp.full((_L,), 1.0, jnp.float32) for _ in range(8)]
            for g in range(8):
                for l in range(_L):
                    col = _splat(pvecs[g], l)
                    glo = plsc.load_gather(chunk_v, [rb_lo + col])
                    ghi = plsc.load_gather(chunk_v, [rb_hi + col])
                    k = l % 4
                    acc[k] = acc[k] * glo
                    acc[4 + k] = acc[4 + k] * ghi
            lo = (acc[0] * acc[1]) * (acc[2] * acc[3])
            hi = (acc[4] * acc[5]) * (acc[6] * acc[7])
            r_idx = jnp.full((_L,), q * 128 + rl, jnp.int32)
            plsc.store_scatter(out_v, [ob_lo + r_idx], lo)
            plsc.store_scatter(out_v, [ob_hi + r_idx], hi)
            return carry2

        lax.fori_loop(0, 128, rule_body, 0)
        return carry

    lax.fori_loop(0, _N_RULES // 128, quarter_body, 0)
    for r in range(_B_PER_W):
        pltpu.sync_copy(out_v.at[pl.ds(r * _OSTRIDE, _N_RULES)],
                        out_hbm.at[pl.ds((b0 + r) * _N_RULES, _N_RULES)])


def kernel(fuzzified_x, input_selectors):
    b = fuzzified_x.shape[0]
    fx_flat = fuzzified_x.reshape(b * _POS)
    sel_t = input_selectors.T
    mesh = plsc.VectorSubcoreMesh(core_axis_name="c", subcore_axis_name="s")
    f = pl.kernel(
        _sc_body,
        out_type=jax.ShapeDtypeStruct((b * _N_RULES,), jnp.float32),
        mesh=mesh,
        compiler_params=pltpu.CompilerParams(needs_layout_passes=False),
        scratch_types=[
            pltpu.VMEM((_B_PER_W * _CSTRIDE,), jnp.float32),
            pltpu.VMEM((128, _N_INPUTS), jnp.float32),
            pltpu.VMEM((_B_PER_W * _OSTRIDE,), jnp.float32),
        ],
    )
    return f(fx_flat, sel_t).reshape(b, _N_RULES)


# speedup vs baseline: 2.3076x; 2.1589x over previous
import jax
import jax.numpy as jnp
from jax import lax
from jax.experimental import pallas as pl
from jax.experimental.pallas import tpu as pltpu
from jax.experimental.pallas import tpu_sc as plsc

_N_MEM = 16
_L = 16
_B_PER_W = 16
_N_INPUTS = 128
_N_RULES = 512
_B_SC = 512
_POS = _N_INPUTS * _N_MEM
_CSTRIDE = _POS + 8
_OSTRIDE = _N_RULES + 8


def _splat(v, l):
    return jnp.take(v, jnp.full((_L,), l, jnp.int32))


def _sc_body(fx_hbm, selt_hbm, out_hbm, chunk_v, sel_v, out_v):
    wid = lax.axis_index("s") * 2 + lax.axis_index("c")
    b0 = wid * _B_PER_W
    for r in range(_B_PER_W):
        pltpu.sync_copy(fx_hbm.at[pl.ds((b0 + r) * _POS, _POS)],
                        chunk_v.at[pl.ds(r * _CSTRIDE, _POS)])
    lane = lax.iota(jnp.int32, _L)
    ones = jnp.full((_L,), 1.0, jnp.float32)
    plsc.store_scatter(chunk_v, [lane * _CSTRIDE + _POS], ones)
    rb = lane * _CSTRIDE
    ob = lane * _OSTRIDE
    half = jnp.full((_L,), 0.5, jnp.float32)

    def quarter_body(q, carry):
        pltpu.sync_copy(selt_hbm.at[pl.ds(q * 128, 128), :], sel_v)

        def rule_body(rl, carry2):
            pvecs = []
            for g in range(8):
                y = sel_v[rl, pl.ds(g * _L, _L)] * jnp.float32(_N_MEM)
                f = y.astype(jnp.int32)
                frac = y - f.astype(jnp.float32)
                m = (f + jnp.where(frac > half, 1, 0)
                     + jnp.where(frac == half, f & 1, 0))
                i_base = (lane + g * _L) * _N_MEM
                pvecs.append(jnp.where(m < _N_MEM, i_base + m, _POS))
            acc = [jnp.full((_L,), 1.0, jnp.float32) for _ in range(8)]
            for g in range(8):
                for l in range(_L):
                    a = rb + _splat(pvecs[g], l)
                    k = (g * _L + l) % 8
                    acc[k] = acc[k] * plsc.load_gather(chunk_v, [a])
            p0 = (acc[0] * acc[1]) * (acc[2] * acc[3])
            p1 = (acc[4] * acc[5]) * (acc[6] * acc[7])
            r_idx = jnp.full((_L,), q * 128 + rl, jnp.int32)
            plsc.store_scatter(out_v, [ob + r_idx], p0 * p1)
            return carry2

        lax.fori_loop(0, 128, rule_body, 0)
        return carry

    lax.fori_loop(0, _N_RULES // 128, quarter_body, 0)
    for r in range(_B_PER_W):
        pltpu.sync_copy(out_v.at[pl.ds(r * _OSTRIDE, _N_RULES)],
                        out_hbm.at[pl.ds((b0 + r) * _N_RULES, _N_RULES)])


def _tc_body(x_ref, sel_ref, out_ref):
    n_inputs, n_rules = sel_ref.shape
    sel = jnp.round(sel_ref[...] * _N_MEM).astype(jnp.int32)
    m_iota = jax.lax.broadcasted_iota(
        jnp.int32, (n_inputs, _N_MEM, n_rules), 1)
    onehot = (sel[:, None, :] == m_iota).astype(jnp.float32)
    onehot = onehot.reshape(n_inputs * _N_MEM, n_rules)
    logx = jnp.maximum(jnp.log(x_ref[...]), jnp.float32(-1e5))
    acc = jax.lax.dot_general(
        logx, onehot, (((1,), (0,)), ((), ())),
        preferred_element_type=jnp.float32,
        precision=jax.lax.Precision.HIGHEST)
    out_ref[...] = jnp.exp(acc)


def kernel(fuzzified_x, input_selectors):
    b = fuzzified_x.shape[0]
    fx_flat = fuzzified_x.reshape(b * _POS)
    sel_t = input_selectors.T

    mesh = plsc.VectorSubcoreMesh(core_axis_name="c", subcore_axis_name="s")
    sc_fn = pl.kernel(
        _sc_body,
        out_type=jax.ShapeDtypeStruct((_B_SC * _N_RULES,), jnp.float32),
        mesh=mesh,
        compiler_params=pltpu.CompilerParams(needs_layout_passes=False),
        scratch_types=[
            pltpu.VMEM((_B_PER_W * _CSTRIDE,), jnp.float32),
            pltpu.VMEM((128, _N_INPUTS), jnp.float32),
            pltpu.VMEM((_B_PER_W * _OSTRIDE,), jnp.float32),
        ],
    )
    out_sc = sc_fn(fx_flat, sel_t).reshape(_B_SC, _N_RULES)

    b_tc = b - _B_SC
    bt = 256
    x2 = fuzzified_x[_B_SC:].reshape(b_tc, _N_INPUTS * _N_MEM)
    out_tc = pl.pallas_call(
        _tc_body,
        grid=(b_tc // bt,),
        in_specs=[
            pl.BlockSpec((bt, _N_INPUTS * _N_MEM), lambda i: (i, 0)),
            pl.BlockSpec((_N_INPUTS, _N_RULES), lambda i: (0, 0)),
        ],
        out_specs=pl.BlockSpec((bt, _N_RULES), lambda i: (i, 0)),
        out_shape=jax.ShapeDtypeStruct((b_tc, _N_RULES), jnp.float32),
    )(x2, input_selectors)

    return jnp.concatenate([out_sc, out_tc], axis=0)


# speedup vs baseline: 2.9083x; 1.2603x over previous
import jax
import jax.numpy as jnp
from jax import lax
from jax.experimental import pallas as pl
from jax.experimental.pallas import tpu as pltpu
from jax.experimental.pallas import tpu_sc as plsc

_N_MEM = 16
_L = 16
_B_PER_W = 16
_N_INPUTS = 128
_N_RULES = 512
_SPLIT = 2
_R_W = _N_RULES // _SPLIT
_B_SC = (32 // _SPLIT) * _B_PER_W
_POS = _N_INPUTS * _N_MEM
_CSTRIDE = _POS + 8
_OSTRIDE = _R_W + 8


def _splat(v, l):
    return jnp.take(v, jnp.full((_L,), l, jnp.int32))


def _sc_body(fx_hbm, selt_hbm, out_hbm, chunk_v, sel_v, out_v):
    wid = lax.axis_index("s") * 2 + lax.axis_index("c")
    b0 = (wid // _SPLIT) * _B_PER_W
    rslice = (wid % _SPLIT) * _R_W
    for r in range(_B_PER_W):
        pltpu.sync_copy(fx_hbm.at[pl.ds((b0 + r) * _POS, _POS)],
                        chunk_v.at[pl.ds(r * _CSTRIDE, _POS)])
    lane = lax.iota(jnp.int32, _L)
    ones = jnp.full((_L,), 1.0, jnp.float32)
    plsc.store_scatter(chunk_v, [lane * _CSTRIDE + _POS], ones)
    rb = lane * _CSTRIDE
    ob = lane * _OSTRIDE
    half = jnp.full((_L,), 0.5, jnp.float32)

    def quarter_body(q, carry):
        pltpu.sync_copy(selt_hbm.at[pl.ds(rslice + q * 128, 128), :], sel_v)

        def rule_body(rl, carry2):
            pvecs = []
            for g in range(8):
                y = sel_v[rl, pl.ds(g * _L, _L)] * jnp.float32(_N_MEM)
                f = y.astype(jnp.int32)
                frac = y - f.astype(jnp.float32)
                m = (f + jnp.where(frac > half, 1, 0)
                     + jnp.where(frac == half, f & 1, 0))
                i_base = (lane + g * _L) * _N_MEM
                pvecs.append(jnp.where(m < _N_MEM, i_base + m, _POS))
            acc = [jnp.full((_L,), 1.0, jnp.float32) for _ in range(8)]
            for g in range(8):
                for l in range(_L):
                    a = rb + _splat(pvecs[g], l)
                    k = (g * _L + l) % 8
                    acc[k] = acc[k] * plsc.load_gather(chunk_v, [a])
            p0 = (acc[0] * acc[1]) * (acc[2] * acc[3])
            p1 = (acc[4] * acc[5]) * (acc[6] * acc[7])
            r_idx = jnp.full((_L,), q * 128 + rl, jnp.int32)
            plsc.store_scatter(out_v, [ob + r_idx], p0 * p1)
            return carry2

        lax.fori_loop(0, 128, rule_body, 0)
        return carry

    lax.fori_loop(0, _R_W // 128, quarter_body, 0)
    for r in range(_B_PER_W):
        pltpu.sync_copy(
            out_v.at[pl.ds(r * _OSTRIDE, _R_W)],
            out_hbm.at[pl.ds((b0 + r) * _N_RULES + rslice, _R_W)])


def _tc_body(x_ref, sel_ref, out_ref):
    n_inputs, n_rules = sel_ref.shape
    sel = jnp.round(sel_ref[...] * _N_MEM).astype(jnp.int32)
    m_iota = jax.lax.broadcasted_iota(
        jnp.int32, (n_inputs, _N_MEM, n_rules), 1)
    onehot = (sel[:, None, :] == m_iota).astype(jnp.float32)
    onehot = onehot.reshape(n_inputs * _N_MEM, n_rules)
    logx = jnp.maximum(jnp.log(x_ref[...]), jnp.float32(-1e5))
    acc = jax.lax.dot_general(
        logx, onehot, (((1,), (0,)), ((), ())),
        preferred_element_type=jnp.float32,
        precision=jax.lax.Precision.HIGHEST)
    out_ref[...] = jnp.exp(acc)


def kernel(fuzzified_x, input_selectors):
    b = fuzzified_x.shape[0]
    fx_flat = fuzzified_x.reshape(b * _POS)
    sel_t = input_selectors.T

    mesh = plsc.VectorSubcoreMesh(core_axis_name="c", subcore_axis_name="s")
    sc_fn = pl.kernel(
        _sc_body,
        out_type=jax.ShapeDtypeStruct((_B_SC * _N_RULES,), jnp.float32),
        mesh=mesh,
        compiler_params=pltpu.CompilerParams(needs_layout_passes=False),
        scratch_types=[
            pltpu.VMEM((_B_PER_W * _CSTRIDE,), jnp.float32),
            pltpu.VMEM((128, _N_INPUTS), jnp.float32),
            pltpu.VMEM((_B_PER_W * _OSTRIDE,), jnp.float32),
        ],
    )
    out_sc = sc_fn(fx_flat, sel_t).reshape(_B_SC, _N_RULES)

    b_tc = b - _B_SC
    bt = 256
    x2 = fuzzified_x[_B_SC:].reshape(b_tc, _N_INPUTS * _N_MEM)
    out_tc = pl.pallas_call(
        _tc_body,
        grid=(b_tc // bt,),
        in_specs=[
            pl.BlockSpec((bt, _N_INPUTS * _N_MEM), lambda i: (i, 0)),
            pl.BlockSpec((_N_INPUTS, _N_RULES), lambda i: (0, 0)),
        ],
        out_specs=pl.BlockSpec((bt, _N_RULES), lambda i: (i, 0)),
        out_shape=jax.ShapeDtypeStruct((b_tc, _N_RULES), jnp.float32),
    )(x2, input_selectors)

    return jnp.concatenate([out_sc, out_tc], axis=0)


# speedup vs baseline: 3.3757x; 1.1607x over previous
import jax
import jax.numpy as jnp
from jax import lax
from jax.experimental import pallas as pl
from jax.experimental.pallas import tpu as pltpu
from jax.experimental.pallas import tpu_sc as plsc

_N_MEM = 16
_L = 16
_B_PER_W = 16
_N_INPUTS = 128
_N_RULES = 512
_SPLIT = 4
_R_W = _N_RULES // _SPLIT
_B_SC = (32 // _SPLIT) * _B_PER_W
_POS = _N_INPUTS * _N_MEM
_CSTRIDE = _POS + 8
_OSTRIDE = _R_W + 8


def _splat(v, l):
    return jnp.take(v, jnp.full((_L,), l, jnp.int32))


def _sc_body(fx_hbm, selt_hbm, out_hbm, chunk_v, sel_v, out_v):
    wid = lax.axis_index("s") * 2 + lax.axis_index("c")
    b0 = (wid // _SPLIT) * _B_PER_W
    rslice = (wid % _SPLIT) * _R_W
    for r in range(_B_PER_W):
        pltpu.sync_copy(fx_hbm.at[pl.ds((b0 + r) * _POS, _POS)],
                        chunk_v.at[pl.ds(r * _CSTRIDE, _POS)])
    lane = lax.iota(jnp.int32, _L)
    ones = jnp.full((_L,), 1.0, jnp.float32)
    plsc.store_scatter(chunk_v, [lane * _CSTRIDE + _POS], ones)
    rb = lane * _CSTRIDE
    ob = lane * _OSTRIDE
    half = jnp.full((_L,), 0.5, jnp.float32)

    def quarter_body(q, carry):
        pltpu.sync_copy(selt_hbm.at[pl.ds(rslice + q * 128, 128), :], sel_v)

        def rule_body(rl, carry2):
            pvecs = []
            for g in range(8):
                y = sel_v[rl, pl.ds(g * _L, _L)] * jnp.float32(_N_MEM)
                f = y.astype(jnp.int32)
                frac = y - f.astype(jnp.float32)
                m = (f + jnp.where(frac > half, 1, 0)
                     + jnp.where(frac == half, f & 1, 0))
                i_base = (lane + g * _L) * _N_MEM
                pvecs.append(jnp.where(m < _N_MEM, i_base + m, _POS))
            acc = [jnp.full((_L,), 1.0, jnp.float32) for _ in range(8)]
            for g in range(8):
                for l in range(_L):
                    a = rb + _splat(pvecs[g], l)
                    k = (g * _L + l) % 8
                    acc[k] = acc[k] * plsc.load_gather(chunk_v, [a])
            p0 = (acc[0] * acc[1]) * (acc[2] * acc[3])
            p1 = (acc[4] * acc[5]) * (acc[6] * acc[7])
            r_idx = jnp.full((_L,), q * 128 + rl, jnp.int32)
            plsc.store_scatter(out_v, [ob + r_idx], p0 * p1)
            return carry2

        lax.fori_loop(0, 128, rule_body, 0)
        return carry

    lax.fori_loop(0, _R_W // 128, quarter_body, 0)
    for r in range(_B_PER_W):
        pltpu.sync_copy(
            out_v.at[pl.ds(r * _OSTRIDE, _R_W)],
            out_hbm.at[pl.ds((b0 + r) * _N_RULES + rslice, _R_W)])


def _tc_body(x_ref, sel_ref, out_ref):
    n_inputs, n_rules = sel_ref.shape
    sel = jnp.round(sel_ref[...] * _N_MEM).astype(jnp.int32)
    m_iota = jax.lax.broadcasted_iota(
        jnp.int32, (n_inputs, _N_MEM, n_rules), 1)
    onehot = (sel[:, None, :] == m_iota).astype(jnp.float32)
    onehot = onehot.reshape(n_inputs * _N_MEM, n_rules)
    logx = jnp.maximum(jnp.log(x_ref[...]), jnp.float32(-1e5))
    acc = jax.lax.dot_general(
        logx, onehot, (((1,), (0,)), ((), ())),
        preferred_element_type=jnp.float32,
        precision=jax.lax.Precision.HIGHEST)
    out_ref[...] = jnp.exp(acc)


def kernel(fuzzified_x, input_selectors):
    b = fuzzified_x.shape[0]
    fx_flat = fuzzified_x.reshape(b * _POS)
    sel_t = input_selectors.T

    mesh = plsc.VectorSubcoreMesh(core_axis_name="c", subcore_axis_name="s")
    sc_fn = pl.kernel(
        _sc_body,
        out_type=jax.ShapeDtypeStruct((_B_SC * _N_RULES,), jnp.float32),
        mesh=mesh,
        compiler_params=pltpu.CompilerParams(needs_layout_passes=False),
        scratch_types=[
            pltpu.VMEM((_B_PER_W * _CSTRIDE,), jnp.float32),
            pltpu.VMEM((128, _N_INPUTS), jnp.float32),
            pltpu.VMEM((_B_PER_W * _OSTRIDE,), jnp.float32),
        ],
    )
    out_sc = sc_fn(fx_flat, sel_t).reshape(_B_SC, _N_RULES)

    b_tc = b - _B_SC
    bt = 448 if b_tc % 448 == 0 else 256
    x2 = fuzzified_x[_B_SC:].reshape(b_tc, _N_INPUTS * _N_MEM)
    out_tc = pl.pallas_call(
        _tc_body,
        grid=(b_tc // bt,),
        in_specs=[
            pl.BlockSpec((bt, _N_INPUTS * _N_MEM), lambda i: (i, 0)),
            pl.BlockSpec((_N_INPUTS, _N_RULES), lambda i: (0, 0)),
        ],
        out_specs=pl.BlockSpec((bt, _N_RULES), lambda i: (i, 0)),
        out_shape=jax.ShapeDtypeStruct((b_tc, _N_RULES), jnp.float32),
    )(x2, input_selectors)

    return jnp.concatenate([out_sc, out_tc], axis=0)


# speedup vs baseline: 4.9573x; 1.4685x over previous
import jax
import jax.numpy as jnp
from jax import lax
from jax.experimental import pallas as pl
from jax.experimental.pallas import tpu as pltpu
from jax.experimental.pallas import tpu_sc as plsc

_N_MEM = 16
_L = 16
_B_PER_W = 16
_N_INPUTS = 128
_N_RULES = 512
_SPLIT = 4
_R_W = _N_RULES // _SPLIT
_B_SC = (32 // _SPLIT) * _B_PER_W
_POS = _N_INPUTS * _N_MEM
_CSTRIDE = _POS + 8
_OSTRIDE = _R_W + 8


def _splat(v, l):
    return jnp.take(v, jnp.full((_L,), l, jnp.int32))


def _sc_body(fx_hbm, selt_hbm, out_hbm, chunk_v, sel_v, out_v):
    wid = lax.axis_index("s") * 2 + lax.axis_index("c")
    b0 = (wid // _SPLIT) * _B_PER_W
    rslice = (wid % _SPLIT) * _R_W
    for r in range(_B_PER_W):
        pltpu.sync_copy(fx_hbm.at[pl.ds((b0 + r) * _POS, _POS)],
                        chunk_v.at[pl.ds(r * _CSTRIDE, _POS)])
    lane = lax.iota(jnp.int32, _L)
    ones = jnp.full((_L,), 1.0, jnp.float32)
    plsc.store_scatter(chunk_v, [lane * _CSTRIDE + _POS], ones)
    rb = lane * _CSTRIDE
    ob = lane * _OSTRIDE
    half = jnp.full((_L,), 0.5, jnp.float32)

    def quarter_body(q, carry):
        pltpu.sync_copy(selt_hbm.at[pl.ds(rslice + q * 128, 128), :], sel_v)

        def rule_body(rl, carry2):
            pvecs = []
            for g in range(8):
                y = sel_v[rl, pl.ds(g * _L, _L)] * jnp.float32(_N_MEM)
                f = y.astype(jnp.int32)
                frac = y - f.astype(jnp.float32)
                m = (f + jnp.where(frac > half, 1, 0)
                     + jnp.where(frac == half, f & 1, 0))
                i_base = (lane + g * _L) * _N_MEM
                pvecs.append(jnp.where(m < _N_MEM, i_base + m, _POS))
            acc = [jnp.full((_L,), 1.0, jnp.float32) for _ in range(8)]
            for g in range(8):
                for l in range(_L):
                    a = rb + _splat(pvecs[g], l)
                    k = (g * _L + l) % 8
                    acc[k] = acc[k] * plsc.load_gather(chunk_v, [a])
            p0 = (acc[0] * acc[1]) * (acc[2] * acc[3])
            p1 = (acc[4] * acc[5]) * (acc[6] * acc[7])
            r_idx = jnp.full((_L,), q * 128 + rl, jnp.int32)
            plsc.store_scatter(out_v, [ob + r_idx], p0 * p1)
            return carry2

        lax.fori_loop(0, 128, rule_body, 0)
        return carry

    lax.fori_loop(0, _R_W // 128, quarter_body, 0)
    for r in range(_B_PER_W):
        pltpu.sync_copy(
            out_v.at[pl.ds(r * _OSTRIDE, _R_W)],
            out_hbm.at[pl.ds((b0 + r) * _N_RULES + rslice, _R_W)])


def _tc_body(x_ref, sel_ref, out_ref):
    n_inputs, n_rules = sel_ref.shape
    sel = jnp.round(sel_ref[...] * _N_MEM).astype(jnp.int32)
    m_iota = jax.lax.broadcasted_iota(
        jnp.int32, (n_inputs, _N_MEM, n_rules), 1)
    onehot = (sel[:, None, :] == m_iota).astype(jnp.float32)
    onehot = onehot.reshape(n_inputs * _N_MEM, n_rules)
    logx = jnp.maximum(jnp.log(x_ref[...]), jnp.float32(-1e5))
    acc = jax.lax.dot_general(
        logx, onehot, (((1,), (0,)), ((), ())),
        preferred_element_type=jnp.float32,
        precision=jax.lax.Precision.HIGHEST)
    out_ref[...] = jnp.exp(acc)


def kernel(fuzzified_x, input_selectors):
    b = fuzzified_x.shape[0]
    fx_flat = fuzzified_x[:_B_SC].reshape(_B_SC * _POS)
    sel_t = input_selectors.T

    mesh = plsc.VectorSubcoreMesh(core_axis_name="c", subcore_axis_name="s")
    sc_fn = pl.kernel(
        _sc_body,
        out_type=jax.ShapeDtypeStruct((_B_SC * _N_RULES,), jnp.float32),
        mesh=mesh,
        compiler_params=pltpu.CompilerParams(needs_layout_passes=False),
        scratch_types=[
            pltpu.VMEM((_B_PER_W * _CSTRIDE,), jnp.float32),
            pltpu.VMEM((128, _N_INPUTS), jnp.float32),
            pltpu.VMEM((_B_PER_W * _OSTRIDE,), jnp.float32),
        ],
    )
    out_sc = sc_fn(fx_flat, sel_t).reshape(_B_SC, _N_RULES)

    b_tc = b - _B_SC
    bt = 448 if b_tc % 448 == 0 else 256
    x2 = fuzzified_x[_B_SC:].reshape(b_tc, _N_INPUTS * _N_MEM)
    out_tc = pl.pallas_call(
        _tc_body,
        grid=(b_tc // bt,),
        in_specs=[
            pl.BlockSpec((bt, _N_INPUTS * _N_MEM), lambda i: (i, 0)),
            pl.BlockSpec((_N_INPUTS, _N_RULES), lambda i: (0, 0)),
        ],
        out_specs=pl.BlockSpec((bt, _N_RULES), lambda i: (i, 0)),
        out_shape=jax.ShapeDtypeStruct((b_tc, _N_RULES), jnp.float32),
    )(x2, input_selectors)

    return jnp.concatenate([out_sc, out_tc], axis=0)
